# Initial kernel scaffold; baseline (speedup 1.0000x reference)
import jax
import jax.numpy as jnp
from jax.experimental import pallas as pl


def kernel(x):
    raise NotImplementedError("write your pallas kernel here")


# speedup vs baseline: 80.3545x; 80.3545x over previous
import jax
import jax.numpy as jnp
from jax.experimental import pallas as pl
from jax.experimental.pallas import tpu as pltpu

_NUM_CLASSES = 80
_CONF = 0.25
_IOU_T = 0.5
_MAX_DET = 300
_MAX_WH = 7680.0
_ROWS = 40
_LANES = 128
_NPAD = _ROWS * _LANES


def _nms_body(x_ref, out_ref, s_ref, ox1_ref, oy1_ref, ox2_ref, oy2_ref,
              area_ref, bx1_ref, by1_ref, bx2_ref, by2_ref, cl_ref, idx_ref):
    obj = x_ref[0, 4]
    valid = obj > _CONF
    best = x_ref[0, 5]
    bidx = jnp.zeros((_ROWS, _LANES), jnp.int32)
    for c in range(1, _NUM_CLASSES):
        p = x_ref[0, 5 + c]
        upd = p > best
        best = jnp.where(upd, p, best)
        bidx = jnp.where(upd, c, bidx)
    cls_f = bidx.astype(jnp.float32)
    score = obj * best

    cx = x_ref[0, 0]
    cy = x_ref[0, 1]
    w2 = x_ref[0, 2] / 2.0
    h2 = x_ref[0, 3] / 2.0
    x1 = cx - w2
    y1 = cy - h2
    x2 = cx + w2
    y2 = cy + h2
    off = cls_f * _MAX_WH
    ox1 = x1 + off
    oy1 = y1 + off
    ox2 = x2 + off
    oy2 = y2 + off
    area = (ox2 - ox1) * (oy2 - oy1)

    s_ref[...] = jnp.where(valid, score, -jnp.inf)
    ox1_ref[...] = ox1
    oy1_ref[...] = oy1
    ox2_ref[...] = ox2
    oy2_ref[...] = oy2
    area_ref[...] = area
    bx1_ref[...] = x1
    by1_ref[...] = y1
    bx2_ref[...] = x2
    by2_ref[...] = y2
    cl_ref[...] = cls_f
    idx_ref[...] = (
        jax.lax.broadcasted_iota(jnp.int32, (_ROWS, _LANES), 0) * _LANES
        + jax.lax.broadcasted_iota(jnp.int32, (_ROWS, _LANES), 1))

    out_ref[...] = jnp.zeros((1, _MAX_DET, 6), jnp.float32)

    def cond(carry):
        cnt, m = carry
        return jnp.logical_and(cnt < _MAX_DET, m > -jnp.inf)

    def body(carry):
        cnt, m = carry
        s = s_ref[...]
        idx = idx_ref[...]
        i = jnp.min(jnp.where(s == m, idx, _NPAD))
        sel = idx == i

        def pick(p):
            return jnp.sum(jnp.where(sel, p, 0.0))

        p_ox1 = pick(ox1_ref[...])
        p_oy1 = pick(oy1_ref[...])
        p_ox2 = pick(ox2_ref[...])
        p_oy2 = pick(oy2_ref[...])
        p_ar = pick(area_ref[...])

        ltx = jnp.maximum(p_ox1, ox1_ref[...])
        lty = jnp.maximum(p_oy1, oy1_ref[...])
        rbx = jnp.minimum(p_ox2, ox2_ref[...])
        rby = jnp.minimum(p_oy2, oy2_ref[...])
        whx = jnp.maximum(rbx - ltx, 0.0)
        why = jnp.maximum(rby - lty, 0.0)
        inter = whx * why
        union = p_ar + area_ref[...] - inter
        iou = inter / union
        supp = iou > _IOU_T

        s_new = jnp.where(jnp.logical_or(supp, sel), -jnp.inf, s)
        s_ref[...] = s_new

        row = jnp.concatenate(
            [jnp.reshape(v, (1, 1)) for v in
             (pick(bx1_ref[...]), pick(by1_ref[...]), pick(bx2_ref[...]),
              pick(by2_ref[...]), m, pick(cl_ref[...]))], axis=1)
        out_ref[0, pl.ds(cnt, 1), :] = row
        return cnt + 1, jnp.max(s_new)

    m0 = jnp.max(s_ref[...])
    jax.lax.while_loop(cond, body, (jnp.int32(0), m0))


def _nms_call(xt, interpret=False):
    batch = xt.shape[0]
    f32 = jnp.float32
    return pl.pallas_call(
        _nms_body,
        grid=(batch,),
        in_specs=[pl.BlockSpec((1, 5 + _NUM_CLASSES, _ROWS, _LANES),
                               lambda b: (b, 0, 0, 0))],
        out_specs=pl.BlockSpec((1, _MAX_DET, 6), lambda b: (b, 0, 0)),
        out_shape=jax.ShapeDtypeStruct((batch, _MAX_DET, 6), f32),
        scratch_shapes=[pltpu.VMEM((_ROWS, _LANES), f32)] * 11
        + [pltpu.VMEM((_ROWS, _LANES), jnp.int32)],
        interpret=interpret,
    )(xt)


def kernel(x):
    n = x.shape[1]
    xp = jnp.pad(x, ((0, 0), (0, _NPAD - n), (0, 0)))
    xt = xp.transpose(0, 2, 1).reshape(x.shape[0], 5 + _NUM_CLASSES,
                                       _ROWS, _LANES)
    return _nms_call(xt)


# speedup vs baseline: 97.9643x; 1.2192x over previous
import jax
import jax.numpy as jnp
from jax.experimental import pallas as pl
from jax.experimental.pallas import tpu as pltpu

_NUM_CLASSES = 80
_CONF = 0.25
_IOU_T = 0.5
_MAX_DET = 300
_MAX_WH = 7680.0
_ROWS = 40
_LANES = 128
_NPAD = _ROWS * _LANES
_BATCH = 4


def _nms_body(x_ref, out_ref, s_ref, ox1_ref, oy1_ref, ox2_ref, oy2_ref,
              area_ref, bx1_ref, by1_ref, bx2_ref, by2_ref, cl_ref, idx_ref):
    obj = x_ref[:, 4]
    valid = obj > _CONF
    best = x_ref[:, 5]
    bidx = jnp.zeros((_BATCH, _ROWS, _LANES), jnp.int32)
    for c in range(1, _NUM_CLASSES):
        p = x_ref[:, 5 + c]
        upd = p > best
        best = jnp.where(upd, p, best)
        bidx = jnp.where(upd, c, bidx)
    cls_f = bidx.astype(jnp.float32)
    score = obj * best

    cx = x_ref[:, 0]
    cy = x_ref[:, 1]
    w2 = x_ref[:, 2] / 2.0
    h2 = x_ref[:, 3] / 2.0
    x1 = cx - w2
    y1 = cy - h2
    x2 = cx + w2
    y2 = cy + h2
    off = cls_f * _MAX_WH
    ox1 = x1 + off
    oy1 = y1 + off
    ox2 = x2 + off
    oy2 = y2 + off
    area = (ox2 - ox1) * (oy2 - oy1)

    s_ref[...] = jnp.where(valid, score, -jnp.inf)
    ox1_ref[...] = ox1
    oy1_ref[...] = oy1
    ox2_ref[...] = ox2
    oy2_ref[...] = oy2
    area_ref[...] = area
    bx1_ref[...] = x1
    by1_ref[...] = y1
    bx2_ref[...] = x2
    by2_ref[...] = y2
    cl_ref[...] = cls_f
    idx_ref[...] = (
        jax.lax.broadcasted_iota(jnp.int32, (_ROWS, _LANES), 0) * _LANES
        + jax.lax.broadcasted_iota(jnp.int32, (_ROWS, _LANES), 1))

    out_ref[...] = jnp.zeros((_BATCH, _MAX_DET, 6), jnp.float32)

    def cond(carry):
        alive = [jnp.logical_and(carry[b] < _MAX_DET,
                                 carry[_BATCH + b] > -jnp.inf)
                 for b in range(_BATCH)]
        a = alive[0]
        for b in range(1, _BATCH):
            a = jnp.logical_or(a, alive[b])
        return a

    def body(carry):
        cnts = carry[:_BATCH]
        ms = carry[_BATCH:]
        idx = idx_ref[...]
        new_cnts = []
        new_ms = []
        for b in range(_BATCH):
            cnt = cnts[b]
            m = ms[b]
            alive = jnp.logical_and(cnt < _MAX_DET, m > -jnp.inf)
            s = s_ref[b]
            i = jnp.min(jnp.where(s == m, idx, _NPAD))
            sel = idx == i

            def pick(p):
                return jnp.sum(jnp.where(sel, p, 0.0))

            p_ox1 = pick(ox1_ref[b])
            p_oy1 = pick(oy1_ref[b])
            p_ox2 = pick(ox2_ref[b])
            p_oy2 = pick(oy2_ref[b])
            p_ar = pick(area_ref[b])

            ltx = jnp.maximum(p_ox1, ox1_ref[b])
            lty = jnp.maximum(p_oy1, oy1_ref[b])
            rbx = jnp.minimum(p_ox2, ox2_ref[b])
            rby = jnp.minimum(p_oy2, oy2_ref[b])
            whx = jnp.maximum(rbx - ltx, 0.0)
            why = jnp.maximum(rby - lty, 0.0)
            inter = whx * why
            union = p_ar + area_ref[b] - inter
            iou = inter / union
            supp = iou > _IOU_T

            kill = jnp.logical_and(alive, jnp.logical_or(supp, sel))
            s_new = jnp.where(kill, -jnp.inf, s)
            s_ref[b] = s_new

            row = jnp.concatenate(
                [jnp.reshape(v, (1, 1)) for v in
                 (pick(bx1_ref[b]), pick(by1_ref[b]), pick(bx2_ref[b]),
                  pick(by2_ref[b]), m, pick(cl_ref[b]))], axis=1)

            @pl.when(alive)
            def _():
                out_ref[b, pl.ds(cnt, 1), :] = row

            new_cnts.append(jnp.where(alive, cnt + 1, cnt))
            new_ms.append(jnp.max(s_new))
        return tuple(new_cnts) + tuple(new_ms)

    m0 = tuple(jnp.max(s_ref[b]) for b in range(_BATCH))
    jax.lax.while_loop(cond, body,
                       tuple(jnp.int32(0) for _ in range(_BATCH)) + m0)


def _nms_call(xt, interpret=False):
    f32 = jnp.float32
    return pl.pallas_call(
        _nms_body,
        out_shape=jax.ShapeDtypeStruct((_BATCH, _MAX_DET, 6), f32),
        scratch_shapes=[pltpu.VMEM((_BATCH, _ROWS, _LANES), f32)] * 11
        + [pltpu.VMEM((_ROWS, _LANES), jnp.int32)],
        interpret=interpret,
    )(xt)


def kernel(x):
    n = x.shape[1]
    xp = jnp.pad(x, ((0, 0), (0, _NPAD - n), (0, 0)))
    xt = xp.transpose(0, 2, 1).reshape(x.shape[0], 5 + _NUM_CLASSES,
                                       _ROWS, _LANES)
    return _nms_call(xt)
